# baseline (speedup 1.0000x reference)
import jax
import jax.numpy as jnp
import numpy as np
from jax.experimental import pallas as pl
from jax.experimental.pallas import tpu as pltpu

_GROUP = 6
_K = 3
_PAD = 1
_DIL = 1
_OFFSET_SCALE = 1.0
_EPS = 1e-6


def _mm_kernel(x_ref, w_ref, b_ref, o_ref):
    o_ref[...] = (
        jnp.dot(x_ref[...], w_ref[...], preferred_element_type=jnp.float32)
        + b_ref[...]
    )


def _out_proj(y2d, W_out, b_out):
    M, C = y2d.shape
    TM = 512
    assert M % TM == 0
    return pl.pallas_call(
        _mm_kernel,
        grid=(M // TM,),
        in_specs=[
            pl.BlockSpec((TM, C), lambda i: (i, 0)),
            pl.BlockSpec((C, C), lambda i: (0, 0)),
            pl.BlockSpec((1, C), lambda i: (0, 0)),
        ],
        out_specs=pl.BlockSpec((TM, C), lambda i: (i, 0)),
        out_shape=jax.ShapeDtypeStruct((M, C), jnp.float32),
    )(y2d, W_out.T, b_out.reshape(1, C))


def _bilinear_sample(xp_g, py, px, Hp, Wp):
    y0 = jnp.floor(py)
    x0 = jnp.floor(px)
    wy = py - y0
    wx = px - x0
    y0i = y0.astype(jnp.int32)
    x0i = x0.astype(jnp.int32)
    N = xp_g.shape[0]
    G = xp_g.shape[3]
    b = jnp.arange(N)[:, None, None, None]
    g = jnp.arange(G)[None, None, None, :]

    def gather(yi, xi):
        valid = ((yi >= 0) & (yi < Hp) & (xi >= 0) & (xi < Wp)).astype(xp_g.dtype)
        yc = jnp.clip(yi, 0, Hp - 1)
        xc = jnp.clip(xi, 0, Wp - 1)
        v = xp_g[b, yc, xc, g]
        return v * valid[..., None]

    v00 = gather(y0i, x0i)
    v01 = gather(y0i, x0i + 1)
    v10 = gather(y0i + 1, x0i)
    v11 = gather(y0i + 1, x0i + 1)
    w00 = ((1 - wy) * (1 - wx))[..., None]
    w01 = ((1 - wy) * wx)[..., None]
    w10 = (wy * (1 - wx))[..., None]
    w11 = (wy * wx)[..., None]
    return v00 * w00 + v01 * w01 + v10 * w10 + v11 * w11


def kernel(x, depth, W_in, b_in, W_dw, b_dw, ln_g, ln_b, W_off, b_off, W_mask, b_mask, W_out, b_out):
    N, H, W, C = x.shape
    G, K, PAD = _GROUP, _K, _PAD
    GC = C // G
    x_proj = x @ W_in.T + b_in
    x1 = jnp.transpose(x, (0, 3, 1, 2))
    x1 = jax.lax.conv_general_dilated(
        x1, W_dw, window_strides=(1, 1),
        padding=((PAD, PAD), (PAD, PAD)), feature_group_count=C,
        dimension_numbers=("NCHW", "OIHW", "NCHW"))
    x1 = x1 + b_dw[None, :, None, None]
    x1 = jnp.transpose(x1, (0, 2, 3, 1))
    mu = jnp.mean(x1, axis=-1, keepdims=True)
    var = jnp.var(x1, axis=-1, keepdims=True)
    x1 = (x1 - mu) / jnp.sqrt(var + _EPS) * ln_g + ln_b
    x1 = jax.nn.gelu(x1, approximate=False)
    offset = x1 @ W_off.T + b_off
    mask_logits = x1 @ W_mask.T + b_mask
    mask = jax.nn.softmax(mask_logits.reshape(N, H, W, G, K * K), axis=-1)
    xp = jnp.pad(x_proj, ((0, 0), (PAD, PAD), (PAD, PAD), (0, 0)))
    Hp, Wp = H + 2 * PAD, W + 2 * PAD
    xp_g = xp.reshape(N, Hp, Wp, G, GC)
    off = offset.reshape(N, H, W, G, K * K, 2) * _OFFSET_SCALE
    pts = (np.arange(K) - (K - 1) // 2) * _DIL
    dy, dx = np.meshgrid(pts, pts, indexing="ij")
    dy = dy.reshape(-1)
    dx = dx.reshape(-1)
    base_y = (jnp.arange(H) + PAD).astype(x.dtype)[None, :, None, None]
    base_x = (jnp.arange(W) + PAD).astype(x.dtype)[None, None, :, None]
    acc = jnp.zeros((N, H, W, G, GC), dtype=x.dtype)
    for k in range(K * K):
        py = base_y + float(dy[k]) + off[..., k, 0]
        px = base_x + float(dx[k]) + off[..., k, 1]
        samp = _bilinear_sample(xp_g, py, px, Hp, Wp)
        acc = acc + samp * mask[..., k][..., None]
    y = acc.reshape(N * H * W, C)
    out = _out_proj(y, W_out, b_out).reshape(N, H, W, C)
    return (out, depth)


# speedup vs baseline: 3.9500x; 3.9500x over previous
import functools

import jax
import jax.numpy as jnp
import numpy as np
from jax import lax
from jax.experimental import pallas as pl
from jax.experimental.pallas import tpu as pltpu
from jax.experimental.pallas import tpu_sc as plsc

_GROUP = 6
_K = 3
_PAD = 1
_DIL = 1
_OFFSET_SCALE = 1.0
_EPS = 1e-6

_NC = 2
_NS = 16
_NW = _NC * _NS


def _mm_kernel(x_ref, w_ref, b_ref, o_ref):
    o_ref[...] = (
        jnp.dot(x_ref[...], w_ref[...], preferred_element_type=jnp.float32)
        + b_ref[...]
    )


def _dense_proj(x2d, Wmat, bvec):
    M, Cin = x2d.shape
    Cout = Wmat.shape[0]
    TM = 512
    assert M % TM == 0
    return pl.pallas_call(
        _mm_kernel,
        grid=(M // TM,),
        in_specs=[
            pl.BlockSpec((TM, Cin), lambda i: (i, 0)),
            pl.BlockSpec((Cin, Cout), lambda i: (0, 0)),
            pl.BlockSpec((1, Cout), lambda i: (0, 0)),
        ],
        out_specs=pl.BlockSpec((TM, Cout), lambda i: (i, 0)),
        out_shape=jax.ShapeDtypeStruct((M, Cout), jnp.float32),
    )(x2d, Wmat.T, bvec.reshape(1, Cout))


_NPTS = 9
_WPAD = 16
_TB = 64
_CSZ = _TB * _NPTS
_CHUNKS = (128, 128, 128, 128, 64)


def _make_sc_gather(n_rows):
    assert n_rows % (_NW * _TB) == 0
    ch = n_rows // _NW
    nt = ch // _TB

    mesh = plsc.VectorSubcoreMesh(core_axis_name="c", subcore_axis_name="s")

    @functools.partial(
        pl.kernel,
        mesh=mesh,
        compiler_params=pltpu.CompilerParams(use_tc_tiling_on_sc=False),
        out_type=jax.ShapeDtypeStruct((n_rows, 16), jnp.float32),
        scratch_types=[
            pltpu.VMEM((4, _CSZ), jnp.int32),
            pltpu.VMEM((4, _TB * _WPAD), jnp.float32),
            pltpu.VMEM((4 * _CSZ, 16), jnp.float32),
            pltpu.VMEM((_TB, 16), jnp.float32),
            pltpu.SemaphoreType.DMA,
        ],
    )
    def sc_gather(tbl, i0, i1, i2, i3, w0, w1, w2, w3, out,
                  idx_v, wgt_v, rows_v, out_v, sem):
        wid = lax.axis_index("s") * _NC + lax.axis_index("c")
        base = wid * ch
        idxs = (i0, i1, i2, i3)
        wgts = (w0, w1, w2, w3)

        def tile_body(t, carry):
            row0 = base + t * _TB
            for c in range(4):
                pltpu.sync_copy(
                    idxs[c].at[pl.ds(row0 * _NPTS, _CSZ)], idx_v.at[c]
                )
                pltpu.sync_copy(
                    wgts[c].at[pl.ds(row0 * _WPAD, _TB * _WPAD)], wgt_v.at[c]
                )
            copies = []
            for c in range(4):
                o = 0
                for sz in _CHUNKS:
                    copies.append(
                        pltpu.async_copy(
                            tbl.at[idx_v.at[c, pl.ds(o, sz)]],
                            rows_v.at[pl.ds(c * _CSZ + o, sz)],
                            sem,
                        )
                    )
                    o += sz
            for cp in copies:
                cp.wait()

            def row_body(r, carry2):
                acc = None
                for c in range(4):
                    wv = wgt_v[c, pl.ds(r * _WPAD, 16)]
                    b0 = c * _CSZ + r * _NPTS
                    for k in range(_NPTS):
                        term = rows_v[b0 + k, :] * wv[k]
                        acc = term if acc is None else acc + term
                out_v[r, :] = acc
                return carry2

            lax.fori_loop(0, _TB, row_body, 0, unroll=False)
            pltpu.sync_copy(out_v, out.at[pl.ds(row0, _TB)])
            return carry

        lax.fori_loop(0, nt, tile_body, 0, unroll=False)

    return sc_gather


def _build_idx_wgt(off, mask, N, H, W, G, Hp, Wp):
    base_y = (jnp.arange(H, dtype=jnp.float32) + _PAD)[None, :, None, None]
    base_x = (jnp.arange(W, dtype=jnp.float32) + _PAD)[None, None, :, None]
    pts = (np.arange(_K) - (_K - 1) // 2) * _DIL
    dy, dx = np.meshgrid(pts, pts, indexing="ij")
    dy = dy.reshape(-1).astype(np.float32)
    dx = dx.reshape(-1).astype(np.float32)

    py = base_y[..., None] + dy + off[..., 0]
    px = base_x[..., None] + dx + off[..., 1]
    y0 = jnp.floor(py)
    x0 = jnp.floor(px)
    wy = py - y0
    wx = px - x0
    y0i = y0.astype(jnp.int32)
    x0i = x0.astype(jnp.int32)

    b_idx = jnp.arange(N, dtype=jnp.int32)[:, None, None, None, None]
    g_idx = jnp.arange(G, dtype=jnp.int32)[None, None, None, :, None]

    idxs = []
    wgts = []
    for cy, cx, w in (
        (0, 0, (1 - wy) * (1 - wx)),
        (0, 1, (1 - wy) * wx),
        (1, 0, wy * (1 - wx)),
        (1, 1, wy * wx),
    ):
        yi = y0i + cy
        xi = x0i + cx
        valid = ((yi >= 0) & (yi < Hp) & (xi >= 0) & (xi < Wp)).astype(
            jnp.float32
        )
        yc = jnp.clip(yi, 0, Hp - 1)
        xc = jnp.clip(xi, 0, Wp - 1)
        idxs.append((((b_idx * Hp + yc) * Wp + xc) * G + g_idx).reshape(-1))
        wg = (w * valid * mask).reshape(-1, _NPTS)
        wg = jnp.pad(wg, ((0, 0), (0, _WPAD - _NPTS))).reshape(-1)
        wgts.append(wg)
    return idxs, wgts


def kernel(x, depth, W_in, b_in, W_dw, b_dw, ln_g, ln_b, W_off, b_off, W_mask, b_mask, W_out, b_out):
    N, H, W, C = x.shape
    G, K, PAD = _GROUP, _K, _PAD
    GC = C // G
    Hp, Wp = H + 2 * PAD, W + 2 * PAD
    M = N * H * W

    x_proj = _dense_proj(x.reshape(M, C), W_in, b_in).reshape(N, H, W, C)

    x1 = jnp.transpose(x, (0, 3, 1, 2))
    x1 = lax.conv_general_dilated(
        x1, W_dw, window_strides=(1, 1),
        padding=((PAD, PAD), (PAD, PAD)), feature_group_count=C,
        dimension_numbers=("NCHW", "OIHW", "NCHW"))
    x1 = x1 + b_dw[None, :, None, None]
    x1 = jnp.transpose(x1, (0, 2, 3, 1))
    mu = jnp.mean(x1, axis=-1, keepdims=True)
    var = jnp.var(x1, axis=-1, keepdims=True)
    x1 = (x1 - mu) / jnp.sqrt(var + _EPS) * ln_g + ln_b
    x1 = jax.nn.gelu(x1, approximate=False)

    offset = _dense_proj(x1.reshape(M, C), W_off, b_off)
    mask_logits = _dense_proj(x1.reshape(M, C), W_mask, b_mask)
    mask = jax.nn.softmax(mask_logits.reshape(N, H, W, G, K * K), axis=-1)

    off = offset.reshape(N, H, W, G, K * K, 2) * _OFFSET_SCALE
    idxs, wgts = _build_idx_wgt(off, mask, N, H, W, G, Hp, Wp)

    xp = jnp.pad(x_proj, ((0, 0), (PAD, PAD), (PAD, PAD), (0, 0)))
    tbl = xp.reshape(N * Hp * Wp * G, GC)

    n_rows = M * G
    y2d = _make_sc_gather(n_rows)(tbl, *idxs, *wgts)

    y = y2d.reshape(M, C)
    out = _dense_proj(y, W_out, b_out).reshape(N, H, W, C)
    return (out, depth)


# speedup vs baseline: 12.2864x; 3.1105x over previous
import functools

import jax
import jax.numpy as jnp
import numpy as np
from jax import lax
from jax.experimental import pallas as pl
from jax.experimental.pallas import tpu as pltpu
from jax.experimental.pallas import tpu_sc as plsc

_GROUP = 6
_K = 3
_PAD = 1
_DIL = 1
_OFFSET_SCALE = 1.0
_EPS = 1e-6

_NC = 2
_NS = 16
_NW = _NC * _NS


def _mm_kernel(x_ref, w_ref, b_ref, o_ref):
    o_ref[...] = (
        jnp.dot(x_ref[...], w_ref[...], preferred_element_type=jnp.float32)
        + b_ref[...]
    )


def _dense_proj(x2d, Wmat, bvec):
    M, Cin = x2d.shape
    Cout = Wmat.shape[0]
    TM = 512
    assert M % TM == 0
    return pl.pallas_call(
        _mm_kernel,
        grid=(M // TM,),
        in_specs=[
            pl.BlockSpec((TM, Cin), lambda i: (i, 0)),
            pl.BlockSpec((Cin, Cout), lambda i: (0, 0)),
            pl.BlockSpec((1, Cout), lambda i: (0, 0)),
        ],
        out_specs=pl.BlockSpec((TM, Cout), lambda i: (i, 0)),
        out_shape=jax.ShapeDtypeStruct((M, Cout), jnp.float32),
    )(x2d, Wmat.T, bvec.reshape(1, Cout))


_NPTS = 9
_TB = 64
_CSZ = _TB * _NPTS
_CHUNKS = (128, 128, 128, 128, 64)


def _make_sc_gather(n_rows):
    assert n_rows % (_NW * _TB) == 0
    ch = n_rows // _NW
    nt = ch // _TB

    mesh = plsc.VectorSubcoreMesh(core_axis_name="c", subcore_axis_name="s")

    @functools.partial(
        pl.kernel,
        mesh=mesh,
        compiler_params=pltpu.CompilerParams(use_tc_tiling_on_sc=False),
        out_type=jax.ShapeDtypeStruct((n_rows, 16), jnp.float32),
        scratch_types=[
            pltpu.VMEM((4, _CSZ), jnp.int32),
            pltpu.VMEM((4, _CSZ + 16), jnp.float32),
            pltpu.VMEM((4 * _CSZ, 16), jnp.float32),
            pltpu.VMEM((_TB, 16), jnp.float32),
            pltpu.SemaphoreType.DMA,
        ],
    )
    def sc_gather(tbl, i0, i1, i2, i3, w0, w1, w2, w3, out,
                  idx_v, wgt_v, rows_v, out_v, sem):
        wid = lax.axis_index("s") * _NC + lax.axis_index("c")
        base = wid * ch
        idxs = (i0, i1, i2, i3)
        wgts = (w0, w1, w2, w3)

        def tile_body(t, carry):
            row0 = base + t * _TB
            for c in range(4):
                pltpu.sync_copy(
                    idxs[c].at[pl.ds(row0 * _NPTS, _CSZ)], idx_v.at[c]
                )
                pltpu.sync_copy(
                    wgts[c].at[pl.ds(row0 * _NPTS, _CSZ)],
                    wgt_v.at[c, pl.ds(0, _CSZ)],
                )
            copies = []
            for c in range(4):
                o = 0
                for sz in _CHUNKS:
                    copies.append(
                        pltpu.async_copy(
                            tbl.at[idx_v.at[c, pl.ds(o, sz)]],
                            rows_v.at[pl.ds(c * _CSZ + o, sz)],
                            sem,
                        )
                    )
                    o += sz
            for cp in copies:
                cp.wait()

            def row_body(r, carry2):
                acc = None
                for c in range(4):
                    wv = wgt_v[c, pl.ds(r * _NPTS, 16)]
                    b0 = c * _CSZ + r * _NPTS
                    for k in range(_NPTS):
                        term = rows_v[b0 + k, :] * wv[k]
                        acc = term if acc is None else acc + term
                out_v[r, :] = acc
                return carry2

            lax.fori_loop(0, _TB, row_body, 0, unroll=False)
            pltpu.sync_copy(out_v, out.at[pl.ds(row0, _TB)])
            return carry

        lax.fori_loop(0, nt, tile_body, 0, unroll=False)

    return sc_gather


_PTM = 512


def _prep_kernel(N, H, W, C, G, Hp, Wp):
    GK = G * _NPTS
    HW = H * W

    def body(x1_ref, lng_ref, lnb_ref, woff_ref, boff_ref, wmsk_ref, bmsk_ref,
             i0, i1, i2, i3, w0, w1, w2, w3):
        x = x1_ref[...]
        mu = jnp.mean(x, axis=1, keepdims=True)
        xc_ = x - mu
        var = jnp.mean(xc_ * xc_, axis=1, keepdims=True)
        xn = xc_ * jax.lax.rsqrt(var + _EPS) * lng_ref[...] + lnb_ref[...]
        xg = 0.5 * xn * (1.0 + jax.lax.erf(xn / np.sqrt(2.0).astype(np.float32)))

        offp = (
            jnp.dot(xg, woff_ref[...], preferred_element_type=jnp.float32)
            + boff_ref[...]
        )
        logits = (
            jnp.dot(xg, wmsk_ref[...], preferred_element_type=jnp.float32)
            + bmsk_ref[...]
        )
        cmax = jnp.max(logits, axis=1, keepdims=True)
        e = jnp.exp(logits - cmax)
        lane = jax.lax.broadcasted_iota(jnp.int32, (GK, G), 0) // _NPTS
        grp = jax.lax.broadcasted_iota(jnp.int32, (GK, G), 1)
        A = (lane == grp).astype(jnp.float32)
        gsum = jnp.dot(e, A, preferred_element_type=jnp.float32)
        denom = jnp.dot(gsum, A.T, preferred_element_type=jnp.float32)
        mask = e / denom

        r_id = pl.program_id(0) * _PTM + jax.lax.broadcasted_iota(
            jnp.int32, (_PTM, 1), 0
        )
        b = r_id // HW
        rem = r_id - b * HW
        h = rem // W
        w_ = rem - h * W
        base_y = (h + _PAD).astype(jnp.float32)
        base_x = (w_ + _PAD).astype(jnp.float32)

        klane = jax.lax.broadcasted_iota(jnp.int32, (1, GK), 1) % _NPTS
        dyv = (klane // _K - (_K - 1) // 2).astype(jnp.float32) * _DIL
        dxv = (klane % _K - (_K - 1) // 2).astype(jnp.float32) * _DIL
        g_lane = jax.lax.broadcasted_iota(jnp.int32, (1, GK), 1) // _NPTS

        py = base_y + dyv + offp[:, :GK]
        px = base_x + dxv + offp[:, GK:]
        y0f = jnp.floor(py)
        x0f = jnp.floor(px)
        wy = py - y0f
        wx = px - x0f
        y0 = y0f.astype(jnp.int32)
        x0 = x0f.astype(jnp.int32)
        tb_base = b * (Hp * Wp * G)

        outs_i = (i0, i1, i2, i3)
        outs_w = (w0, w1, w2, w3)
        corners = (
            (0, 0, (1 - wy) * (1 - wx)),
            (0, 1, (1 - wy) * wx),
            (1, 0, wy * (1 - wx)),
            (1, 1, wy * wx),
        )
        for c, (cy, cx, wgt) in enumerate(corners):
            yi = y0 + cy
            xi = x0 + cx
            valid = (yi >= 0) & (yi < Hp) & (xi >= 0) & (xi < Wp)
            ycl = jnp.clip(yi, 0, Hp - 1)
            xcl = jnp.clip(xi, 0, Wp - 1)
            outs_i[c][...] = tb_base + (ycl * Wp + xcl) * G + g_lane
            outs_w[c][...] = wgt * valid.astype(jnp.float32) * mask

    M = N * H * W
    grid = (M // _PTM,)
    row_spec = pl.BlockSpec((_PTM, C), lambda i: (i, 0))
    fix = lambda shp: pl.BlockSpec(shp, lambda i: tuple(0 for _ in shp))
    out54 = pl.BlockSpec((_PTM, GK), lambda i: (i, 0))

    def run(x1_2d, ln_g, ln_b, W_offp, b_offp, W_maskT, b_mask):
        return pl.pallas_call(
            body,
            grid=grid,
            in_specs=[
                row_spec,
                fix((1, C)),
                fix((1, C)),
                fix((C, 2 * GK)),
                fix((1, 2 * GK)),
                fix((C, GK)),
                fix((1, GK)),
            ],
            out_specs=tuple([out54] * 8),
            out_shape=tuple(
                [jax.ShapeDtypeStruct((M, GK), jnp.int32)] * 4
                + [jax.ShapeDtypeStruct((M, GK), jnp.float32)] * 4
            ),
        )(x1_2d, ln_g.reshape(1, C), ln_b.reshape(1, C), W_offp,
          b_offp.reshape(1, 2 * GK), W_maskT, b_mask.reshape(1, GK))

    return run


def kernel(x, depth, W_in, b_in, W_dw, b_dw, ln_g, ln_b, W_off, b_off, W_mask, b_mask, W_out, b_out):
    N, H, W, C = x.shape
    G, K, PAD = _GROUP, _K, _PAD
    GC = C // G
    Hp, Wp = H + 2 * PAD, W + 2 * PAD
    M = N * H * W

    x_proj = _dense_proj(x.reshape(M, C), W_in, b_in).reshape(N, H, W, C)

    x1 = jnp.transpose(x, (0, 3, 1, 2))
    x1 = lax.conv_general_dilated(
        x1, W_dw, window_strides=(1, 1),
        padding=((PAD, PAD), (PAD, PAD)), feature_group_count=C,
        dimension_numbers=("NCHW", "OIHW", "NCHW"))
    x1 = x1 + b_dw[None, :, None, None]
    x1 = jnp.transpose(x1, (0, 2, 3, 1))

    GK = G * _NPTS
    perm = np.concatenate(
        [np.arange(GK) * 2, np.arange(GK) * 2 + 1]
    )
    W_offp = W_off[perm].T * _OFFSET_SCALE
    b_offp = b_off[perm] * _OFFSET_SCALE

    prep = _prep_kernel(N, H, W, C, G, Hp, Wp)
    i0, i1, i2, i3, w0, w1, w2, w3 = prep(
        x1.reshape(M, C), ln_g, ln_b, W_offp, b_offp, W_mask.T, b_mask
    )
    idxs = [a.reshape(-1) for a in (i0, i1, i2, i3)]
    wgts = [a.reshape(-1) for a in (w0, w1, w2, w3)]

    xp = jnp.pad(x_proj, ((0, 0), (PAD, PAD), (PAD, PAD), (0, 0)))
    tbl = xp.reshape(N * Hp * Wp * G, GC)

    n_rows = M * G
    y2d = _make_sc_gather(n_rows)(tbl, *idxs, *wgts)

    y = y2d.reshape(M, C)
    out = _dense_proj(y, W_out, b_out).reshape(N, H, W, C)
    return (out, depth)


# speedup vs baseline: 21.6928x; 1.7656x over previous
import functools

import jax
import jax.numpy as jnp
import numpy as np
from jax import lax
from jax.experimental import pallas as pl
from jax.experimental.pallas import tpu as pltpu
from jax.experimental.pallas import tpu_sc as plsc

_GROUP = 6
_K = 3
_PAD = 1
_DIL = 1
_OFFSET_SCALE = 1.0
_EPS = 1e-6

_NC = 2
_NS = 16
_NW = _NC * _NS


def _mm_kernel(x_ref, w_ref, b_ref, o_ref):
    o_ref[...] = (
        jnp.dot(x_ref[...], w_ref[...], preferred_element_type=jnp.float32)
        + b_ref[...]
    )


def _dense_proj(x2d, Wmat, bvec):
    M, Cin = x2d.shape
    Cout = Wmat.shape[0]
    TM = 512
    assert M % TM == 0
    return pl.pallas_call(
        _mm_kernel,
        grid=(M // TM,),
        in_specs=[
            pl.BlockSpec((TM, Cin), lambda i: (i, 0)),
            pl.BlockSpec((Cin, Cout), lambda i: (0, 0)),
            pl.BlockSpec((1, Cout), lambda i: (0, 0)),
        ],
        out_specs=pl.BlockSpec((TM, Cout), lambda i: (i, 0)),
        out_shape=jax.ShapeDtypeStruct((M, Cout), jnp.float32),
    )(x2d, Wmat.T, bvec.reshape(1, Cout))


_NPTS = 9
_TB = 64
_CSZ = _TB * _NPTS
_CHUNKS = (128, 128, 128, 128, 64)


def _make_sc_gather(n_rows):
    assert n_rows % (_NW * _TB) == 0
    ch = n_rows // _NW
    nt = ch // _TB

    mesh = plsc.VectorSubcoreMesh(core_axis_name="c", subcore_axis_name="s")

    @functools.partial(
        pl.kernel,
        mesh=mesh,
        compiler_params=pltpu.CompilerParams(use_tc_tiling_on_sc=False),
        out_type=jax.ShapeDtypeStruct((n_rows, 16), jnp.float32),
        scratch_types=[
            pltpu.VMEM((2, 4, _CSZ), jnp.int32),
            pltpu.VMEM((2, 4, _CSZ + 16), jnp.float32),
            pltpu.VMEM((2, 4 * _CSZ, 16), jnp.float32),
            pltpu.VMEM((2, _TB, 16), jnp.float32),
            pltpu.SemaphoreType.DMA,
            pltpu.SemaphoreType.DMA,
            pltpu.SemaphoreType.DMA,
            pltpu.SemaphoreType.DMA,
        ],
    )
    def sc_gather(tbl, i0, i1, i2, i3, w0, w1, w2, w3, out,
                  idx_v, wgt_v, rows_v, out_v, isem, wsem, gsem, osem):
        wid = lax.axis_index("s") * _NC + lax.axis_index("c")
        base = wid * ch
        idxs = (i0, i1, i2, i3)
        wgts = (w0, w1, w2, w3)

        def issue_idx(t, bf):
            for c in range(4):
                pltpu.async_copy(
                    idxs[c].at[pl.ds((base + t * _TB) * _NPTS, _CSZ)],
                    idx_v.at[bf, c], isem)

        def wait_idx(bf):
            for c in range(4):
                pltpu.make_async_copy(
                    idxs[c].at[pl.ds(0, _CSZ)], idx_v.at[bf, c], isem).wait()

        def issue_wgt(t, bf):
            for c in range(4):
                pltpu.async_copy(
                    wgts[c].at[pl.ds((base + t * _TB) * _NPTS, _CSZ)],
                    wgt_v.at[bf, c, pl.ds(0, _CSZ)], wsem)

        def wait_wgt(bf):
            for c in range(4):
                pltpu.make_async_copy(
                    wgts[c].at[pl.ds(0, _CSZ)],
                    wgt_v.at[bf, c, pl.ds(0, _CSZ)], wsem).wait()

        def fire_gathers(bf):
            for c in range(4):
                o = 0
                for sz in _CHUNKS:
                    pltpu.async_copy(
                        tbl.at[idx_v.at[bf, c, pl.ds(o, sz)]],
                        rows_v.at[bf, pl.ds(c * _CSZ + o, sz)], gsem)
                    o += sz

        def wait_gathers(bf):
            for c in range(4):
                o = 0
                for sz in _CHUNKS:
                    pltpu.make_async_copy(
                        tbl.at[pl.ds(0, sz)],
                        rows_v.at[bf, pl.ds(c * _CSZ + o, sz)], gsem).wait()
                    o += sz

        def wait_out(bf):
            pltpu.make_async_copy(
                out_v.at[bf], out.at[pl.ds(0, _TB)], osem).wait()

        issue_idx(0, 0)
        wait_idx(0)
        fire_gathers(0)
        issue_idx(1, 1)
        issue_wgt(0, 0)

        def step(s, carry):
            for b in range(2):
                t = s * 2 + b
                bf = b
                nbf = 1 - b

                @pl.when(t + 1 < nt)
                def _():
                    wait_idx(nbf)
                    fire_gathers(nbf)

                wait_gathers(bf)

                @pl.when(t + 2 < nt)
                def _():
                    issue_idx(t + 2, bf)

                wait_wgt(bf)

                @pl.when(t + 1 < nt)
                def _():
                    issue_wgt(t + 1, nbf)

                @pl.when(t >= 2)
                def _():
                    wait_out(bf)

                def row_body(r, carry2):
                    acc = None
                    for c in range(4):
                        wv = wgt_v[bf, c, pl.ds(r * _NPTS, 16)]
                        b0 = c * _CSZ + r * _NPTS
                        for k in range(_NPTS):
                            term = rows_v[bf, b0 + k, :] * wv[k]
                            acc = term if acc is None else acc + term
                    out_v[bf, r, :] = acc
                    return carry2

                lax.fori_loop(0, _TB, row_body, 0, unroll=False)
                pltpu.async_copy(
                    out_v.at[bf], out.at[pl.ds(base + t * _TB, _TB)], osem)
            return carry

        lax.fori_loop(0, nt // 2, step, 0, unroll=False)
        wait_out(0)
        wait_out(1)

    return sc_gather


_PTM = 512


def _prep_kernel(N, H, W, C, G, Hp, Wp):
    GK = G * _NPTS
    HW = H * W

    def body(x1_ref, lng_ref, lnb_ref, woff_ref, boff_ref, wmsk_ref, bmsk_ref,
             i0, i1, i2, i3, w0, w1, w2, w3):
        x = x1_ref[...]
        mu = jnp.mean(x, axis=1, keepdims=True)
        xc_ = x - mu
        var = jnp.mean(xc_ * xc_, axis=1, keepdims=True)
        xn = xc_ * jax.lax.rsqrt(var + _EPS) * lng_ref[...] + lnb_ref[...]
        xg = 0.5 * xn * (1.0 + jax.lax.erf(xn / np.sqrt(2.0).astype(np.float32)))

        offp = (
            jnp.dot(xg, woff_ref[...], preferred_element_type=jnp.float32)
            + boff_ref[...]
        )
        logits = (
            jnp.dot(xg, wmsk_ref[...], preferred_element_type=jnp.float32)
            + bmsk_ref[...]
        )
        cmax = jnp.max(logits, axis=1, keepdims=True)
        e = jnp.exp(logits - cmax)
        lane = jax.lax.broadcasted_iota(jnp.int32, (GK, G), 0) // _NPTS
        grp = jax.lax.broadcasted_iota(jnp.int32, (GK, G), 1)
        A = (lane == grp).astype(jnp.float32)
        gsum = jnp.dot(e, A, preferred_element_type=jnp.float32)
        denom = jnp.dot(gsum, A.T, preferred_element_type=jnp.float32)
        mask = e / denom

        r_id = pl.program_id(0) * _PTM + jax.lax.broadcasted_iota(
            jnp.int32, (_PTM, 1), 0
        )
        b = r_id // HW
        rem = r_id - b * HW
        h = rem // W
        w_ = rem - h * W
        base_y = (h + _PAD).astype(jnp.float32)
        base_x = (w_ + _PAD).astype(jnp.float32)

        klane = jax.lax.broadcasted_iota(jnp.int32, (1, GK), 1) % _NPTS
        dyv = (klane // _K - (_K - 1) // 2).astype(jnp.float32) * _DIL
        dxv = (klane % _K - (_K - 1) // 2).astype(jnp.float32) * _DIL
        g_lane = jax.lax.broadcasted_iota(jnp.int32, (1, GK), 1) // _NPTS

        py = base_y + dyv + offp[:, :GK]
        px = base_x + dxv + offp[:, GK:]
        y0f = jnp.floor(py)
        x0f = jnp.floor(px)
        wy = py - y0f
        wx = px - x0f
        y0 = y0f.astype(jnp.int32)
        x0 = x0f.astype(jnp.int32)
        tb_base = b * (Hp * Wp * G)

        outs_i = (i0, i1, i2, i3)
        outs_w = (w0, w1, w2, w3)
        corners = (
            (0, 0, (1 - wy) * (1 - wx)),
            (0, 1, (1 - wy) * wx),
            (1, 0, wy * (1 - wx)),
            (1, 1, wy * wx),
        )
        for c, (cy, cx, wgt) in enumerate(corners):
            yi = y0 + cy
            xi = x0 + cx
            valid = (yi >= 0) & (yi < Hp) & (xi >= 0) & (xi < Wp)
            ycl = jnp.clip(yi, 0, Hp - 1)
            xcl = jnp.clip(xi, 0, Wp - 1)
            outs_i[c][...] = tb_base + (ycl * Wp + xcl) * G + g_lane
            outs_w[c][...] = wgt * valid.astype(jnp.float32) * mask

    M = N * H * W
    grid = (M // _PTM,)
    row_spec = pl.BlockSpec((_PTM, C), lambda i: (i, 0))
    fix = lambda shp: pl.BlockSpec(shp, lambda i: tuple(0 for _ in shp))
    out54 = pl.BlockSpec((_PTM, GK), lambda i: (i, 0))

    def run(x1_2d, ln_g, ln_b, W_offp, b_offp, W_maskT, b_mask):
        return pl.pallas_call(
            body,
            grid=grid,
            in_specs=[
                row_spec,
                fix((1, C)),
                fix((1, C)),
                fix((C, 2 * GK)),
                fix((1, 2 * GK)),
                fix((C, GK)),
                fix((1, GK)),
            ],
            out_specs=tuple([out54] * 8),
            out_shape=tuple(
                [jax.ShapeDtypeStruct((M, GK), jnp.int32)] * 4
                + [jax.ShapeDtypeStruct((M, GK), jnp.float32)] * 4
            ),
        )(x1_2d, ln_g.reshape(1, C), ln_b.reshape(1, C), W_offp,
          b_offp.reshape(1, 2 * GK), W_maskT, b_mask.reshape(1, GK))

    return run


def kernel(x, depth, W_in, b_in, W_dw, b_dw, ln_g, ln_b, W_off, b_off, W_mask, b_mask, W_out, b_out):
    N, H, W, C = x.shape
    G, K, PAD = _GROUP, _K, _PAD
    GC = C // G
    Hp, Wp = H + 2 * PAD, W + 2 * PAD
    M = N * H * W

    x_proj = _dense_proj(x.reshape(M, C), W_in, b_in).reshape(N, H, W, C)

    x1 = jnp.transpose(x, (0, 3, 1, 2))
    x1 = lax.conv_general_dilated(
        x1, W_dw, window_strides=(1, 1),
        padding=((PAD, PAD), (PAD, PAD)), feature_group_count=C,
        dimension_numbers=("NCHW", "OIHW", "NCHW"))
    x1 = x1 + b_dw[None, :, None, None]
    x1 = jnp.transpose(x1, (0, 2, 3, 1))

    GK = G * _NPTS
    perm = np.concatenate(
        [np.arange(GK) * 2, np.arange(GK) * 2 + 1]
    )
    W_offp = W_off[perm].T * _OFFSET_SCALE
    b_offp = b_off[perm] * _OFFSET_SCALE

    prep = _prep_kernel(N, H, W, C, G, Hp, Wp)
    i0, i1, i2, i3, w0, w1, w2, w3 = prep(
        x1.reshape(M, C), ln_g, ln_b, W_offp, b_offp, W_mask.T, b_mask
    )
    idxs = [a.reshape(-1) for a in (i0, i1, i2, i3)]
    wgts = [a.reshape(-1) for a in (w0, w1, w2, w3)]

    xp = jnp.pad(x_proj, ((0, 0), (PAD, PAD), (PAD, PAD), (0, 0)))
    tbl = xp.reshape(N * Hp * Wp * G, GC)

    n_rows = M * G
    y2d = _make_sc_gather(n_rows)(tbl, *idxs, *wgts)

    y = y2d.reshape(M, C)
    out = _dense_proj(y, W_out, b_out).reshape(N, H, W, C)
    return (out, depth)


# speedup vs baseline: 21.6960x; 1.0001x over previous
import functools

import jax
import jax.numpy as jnp
import numpy as np
from jax import lax
from jax.experimental import pallas as pl
from jax.experimental.pallas import tpu as pltpu
from jax.experimental.pallas import tpu_sc as plsc

_GROUP = 6
_K = 3
_PAD = 1
_DIL = 1
_OFFSET_SCALE = 1.0
_EPS = 1e-6

_NC = 2
_NS = 16
_NW = _NC * _NS


def _mm_kernel(x_ref, w_ref, b_ref, o_ref):
    o_ref[...] = (
        jnp.dot(x_ref[...], w_ref[...], preferred_element_type=jnp.float32)
        + b_ref[...]
    )


def _dense_proj(x2d, Wmat, bvec):
    M, Cin = x2d.shape
    Cout = Wmat.shape[0]
    TM = 512
    assert M % TM == 0
    return pl.pallas_call(
        _mm_kernel,
        grid=(M // TM,),
        in_specs=[
            pl.BlockSpec((TM, Cin), lambda i: (i, 0)),
            pl.BlockSpec((Cin, Cout), lambda i: (0, 0)),
            pl.BlockSpec((1, Cout), lambda i: (0, 0)),
        ],
        out_specs=pl.BlockSpec((TM, Cout), lambda i: (i, 0)),
        out_shape=jax.ShapeDtypeStruct((M, Cout), jnp.float32),
    )(x2d, Wmat.T, bvec.reshape(1, Cout))


_NPTS = 9
_TB = 64
_CSZ = _TB * _NPTS
_CHUNKS = (128, 128, 128, 128, 64)


def _make_sc_gather(n_rows):
    assert n_rows % (_NW * _TB) == 0
    ch = n_rows // _NW
    nt = ch // _TB

    mesh = plsc.VectorSubcoreMesh(core_axis_name="c", subcore_axis_name="s")

    @functools.partial(
        pl.kernel,
        mesh=mesh,
        compiler_params=pltpu.CompilerParams(use_tc_tiling_on_sc=False),
        out_type=jax.ShapeDtypeStruct((n_rows, 16), jnp.float32),
        scratch_types=[
            pltpu.VMEM((2, 4, _CSZ), jnp.int32),
            pltpu.VMEM((2, 4, _CSZ + 16), jnp.float32),
            pltpu.VMEM((2, 4 * _CSZ, 16), jnp.float32),
            pltpu.VMEM((2, _TB, 16), jnp.float32),
            pltpu.SemaphoreType.DMA,
            pltpu.SemaphoreType.DMA,
            pltpu.SemaphoreType.DMA,
            pltpu.SemaphoreType.DMA,
        ],
    )
    def sc_gather(tbl, i0, i1, i2, i3, w0, w1, w2, w3, out,
                  idx_v, wgt_v, rows_v, out_v, isem, wsem, gsem, osem):
        wid = lax.axis_index("s") * _NC + lax.axis_index("c")
        base = wid * ch
        idxs = (i0, i1, i2, i3)
        wgts = (w0, w1, w2, w3)

        def issue_idx(t, bf):
            for c in range(4):
                pltpu.async_copy(
                    idxs[c].at[pl.ds((base + t * _TB) * _NPTS, _CSZ)],
                    idx_v.at[bf, c], isem)

        def wait_idx(bf):
            for c in range(4):
                pltpu.make_async_copy(
                    idxs[c].at[pl.ds(0, _CSZ)], idx_v.at[bf, c], isem).wait()

        def issue_wgt(t, bf):
            for c in range(4):
                pltpu.async_copy(
                    wgts[c].at[pl.ds((base + t * _TB) * _NPTS, _CSZ)],
                    wgt_v.at[bf, c, pl.ds(0, _CSZ)], wsem)

        def wait_wgt(bf):
            for c in range(4):
                pltpu.make_async_copy(
                    wgts[c].at[pl.ds(0, _CSZ)],
                    wgt_v.at[bf, c, pl.ds(0, _CSZ)], wsem).wait()

        def fire_gathers(bf):
            for c in range(4):
                o = 0
                for sz in _CHUNKS:
                    pltpu.async_copy(
                        tbl.at[idx_v.at[bf, c, pl.ds(o, sz)]],
                        rows_v.at[bf, pl.ds(c * _CSZ + o, sz)], gsem)
                    o += sz

        def wait_gathers(bf):
            for c in range(4):
                o = 0
                for sz in _CHUNKS:
                    pltpu.make_async_copy(
                        tbl.at[pl.ds(0, sz)],
                        rows_v.at[bf, pl.ds(c * _CSZ + o, sz)], gsem).wait()
                    o += sz

        def wait_out(bf):
            pltpu.make_async_copy(
                out_v.at[bf], out.at[pl.ds(0, _TB)], osem).wait()

        issue_idx(0, 0)
        wait_idx(0)
        fire_gathers(0)
        issue_idx(1, 1)
        issue_wgt(0, 0)

        def step(s, carry):
            for b in range(2):
                t = s * 2 + b
                bf = b
                nbf = 1 - b

                @pl.when(t + 1 < nt)
                def _():
                    wait_idx(nbf)
                    fire_gathers(nbf)

                wait_gathers(bf)

                @pl.when(t + 2 < nt)
                def _():
                    issue_idx(t + 2, bf)

                wait_wgt(bf)

                @pl.when(t + 1 < nt)
                def _():
                    issue_wgt(t + 1, nbf)

                @pl.when(t >= 2)
                def _():
                    wait_out(bf)

                def row_body(r, carry2):
                    acc = None
                    for c in range(4):
                        wv = wgt_v[bf, c, pl.ds(r * _NPTS, 16)]
                        b0 = c * _CSZ + r * _NPTS
                        for k in range(_NPTS):
                            term = rows_v[bf, b0 + k, :] * wv[k]
                            acc = term if acc is None else acc + term
                    out_v[bf, r, :] = acc
                    return carry2

                lax.fori_loop(0, _TB, row_body, 0, unroll=False)
                pltpu.async_copy(
                    out_v.at[bf], out.at[pl.ds(base + t * _TB, _TB)], osem)
            return carry

        lax.fori_loop(0, nt // 2, step, 0, unroll=False)
        wait_out(0)
        wait_out(1)

    return sc_gather


_PTM = 512


def _prep_kernel(N, H, W, C, G, Hp, Wp):
    GK = G * _NPTS
    HW = H * W

    def body(x1_ref, lng_ref, lnb_ref, woff_ref, boff_ref, wmsk_ref, bmsk_ref,
             i0, i1, i2, i3, w0, w1, w2, w3):
        x = x1_ref[...]
        mu = jnp.mean(x, axis=1, keepdims=True)
        xc_ = x - mu
        var = jnp.mean(xc_ * xc_, axis=1, keepdims=True)
        xn = xc_ * jax.lax.rsqrt(var + _EPS) * lng_ref[...] + lnb_ref[...]
        xg = 0.5 * xn * (1.0 + jax.lax.erf(xn / np.sqrt(2.0).astype(np.float32)))

        offp = (
            jnp.dot(xg, woff_ref[...], preferred_element_type=jnp.float32)
            + boff_ref[...]
        )
        logits = (
            jnp.dot(xg, wmsk_ref[...], preferred_element_type=jnp.float32)
            + bmsk_ref[...]
        )
        cmax = jnp.max(logits, axis=1, keepdims=True)
        e = jnp.exp(logits - cmax)
        lane = jax.lax.broadcasted_iota(jnp.int32, (GK, G), 0) // _NPTS
        grp = jax.lax.broadcasted_iota(jnp.int32, (GK, G), 1)
        A = (lane == grp).astype(jnp.float32)
        gsum = jnp.dot(e, A, preferred_element_type=jnp.float32)
        denom = jnp.dot(gsum, A.T, preferred_element_type=jnp.float32)
        mask = e / denom

        r_id = pl.program_id(0) * _PTM + jax.lax.broadcasted_iota(
            jnp.int32, (_PTM, 1), 0
        )
        b = r_id // HW
        rem = r_id - b * HW
        h = rem // W
        w_ = rem - h * W
        base_y = (h + _PAD).astype(jnp.float32)
        base_x = (w_ + _PAD).astype(jnp.float32)

        klane = jax.lax.broadcasted_iota(jnp.int32, (1, GK), 1) % _NPTS
        dyv = (klane // _K - (_K - 1) // 2).astype(jnp.float32) * _DIL
        dxv = (klane % _K - (_K - 1) // 2).astype(jnp.float32) * _DIL
        g_lane = jax.lax.broadcasted_iota(jnp.int32, (1, GK), 1) // _NPTS

        py = base_y + dyv + offp[:, :GK]
        px = base_x + dxv + offp[:, GK:]
        y0f = jnp.floor(py)
        x0f = jnp.floor(px)
        wy = py - y0f
        wx = px - x0f
        y0 = y0f.astype(jnp.int32)
        x0 = x0f.astype(jnp.int32)
        tb_base = b * (Hp * Wp * G)

        outs_i = (i0, i1, i2, i3)
        outs_w = (w0, w1, w2, w3)
        corners = (
            (0, 0, (1 - wy) * (1 - wx)),
            (0, 1, (1 - wy) * wx),
            (1, 0, wy * (1 - wx)),
            (1, 1, wy * wx),
        )
        for c, (cy, cx, wgt) in enumerate(corners):
            yi = y0 + cy
            xi = x0 + cx
            valid = (yi >= 0) & (yi < Hp) & (xi >= 0) & (xi < Wp)
            ycl = jnp.clip(yi, 0, Hp - 1)
            xcl = jnp.clip(xi, 0, Wp - 1)
            outs_i[c][...] = tb_base + (ycl * Wp + xcl) * G + g_lane
            outs_w[c][...] = wgt * valid.astype(jnp.float32) * mask

    M = N * H * W
    grid = (M // _PTM,)
    row_spec = pl.BlockSpec((_PTM, C), lambda i: (i, 0))
    fix = lambda shp: pl.BlockSpec(shp, lambda i: tuple(0 for _ in shp))
    out54 = pl.BlockSpec((_PTM, GK), lambda i: (i, 0))

    def run(x1_2d, ln_g, ln_b, W_offp, b_offp, W_maskT, b_mask):
        return pl.pallas_call(
            body,
            grid=grid,
            in_specs=[
                row_spec,
                fix((1, C)),
                fix((1, C)),
                fix((C, 2 * GK)),
                fix((1, 2 * GK)),
                fix((C, GK)),
                fix((1, GK)),
            ],
            out_specs=tuple([out54] * 8),
            out_shape=tuple(
                [jax.ShapeDtypeStruct((M, GK), jnp.int32)] * 4
                + [jax.ShapeDtypeStruct((M, GK), jnp.float32)] * 4
            ),
        )(x1_2d, ln_g.reshape(1, C), ln_b.reshape(1, C), W_offp,
          b_offp.reshape(1, 2 * GK), W_maskT, b_mask.reshape(1, GK))

    return run


def kernel(x, depth, W_in, b_in, W_dw, b_dw, ln_g, ln_b, W_off, b_off, W_mask, b_mask, W_out, b_out):
    N, H, W, C = x.shape
    G, K, PAD = _GROUP, _K, _PAD
    GC = C // G
    Hp, Wp = H + 2 * PAD, W + 2 * PAD
    M = N * H * W

    x_proj = _dense_proj(x.reshape(M, C), W_in, b_in).reshape(N, H, W, C)

    w_hwio = jnp.transpose(W_dw, (2, 3, 1, 0))
    x1 = lax.conv_general_dilated(
        x, w_hwio, window_strides=(1, 1),
        padding=((PAD, PAD), (PAD, PAD)), feature_group_count=C,
        dimension_numbers=("NHWC", "HWIO", "NHWC"))
    x1 = x1 + b_dw

    GK = G * _NPTS
    perm = np.concatenate(
        [np.arange(GK) * 2, np.arange(GK) * 2 + 1]
    )
    W_offp = W_off[perm].T * _OFFSET_SCALE
    b_offp = b_off[perm] * _OFFSET_SCALE

    prep = _prep_kernel(N, H, W, C, G, Hp, Wp)
    i0, i1, i2, i3, w0, w1, w2, w3 = prep(
        x1.reshape(M, C), ln_g, ln_b, W_offp, b_offp, W_mask.T, b_mask
    )
    idxs = [a.reshape(-1) for a in (i0, i1, i2, i3)]
    wgts = [a.reshape(-1) for a in (w0, w1, w2, w3)]

    xp = jnp.pad(x_proj, ((0, 0), (PAD, PAD), (PAD, PAD), (0, 0)))
    tbl = xp.reshape(N * Hp * Wp * G, GC)

    n_rows = M * G
    y2d = _make_sc_gather(n_rows)(tbl, *idxs, *wgts)

    y = y2d.reshape(M, C)
    out = _dense_proj(y, W_out, b_out).reshape(N, H, W, C)
    return (out, depth)


# speedup vs baseline: 21.7283x; 1.0015x over previous
import functools

import jax
import jax.numpy as jnp
import numpy as np
from jax import lax
from jax.experimental import pallas as pl
from jax.experimental.pallas import tpu as pltpu
from jax.experimental.pallas import tpu_sc as plsc

_GROUP = 6
_K = 3
_PAD = 1
_DIL = 1
_OFFSET_SCALE = 1.0
_EPS = 1e-6

_NC = 2
_NS = 16
_NW = _NC * _NS


def _mm_kernel(x_ref, w_ref, b_ref, o_ref):
    o_ref[...] = (
        jnp.dot(x_ref[...], w_ref[...], preferred_element_type=jnp.float32)
        + b_ref[...]
    )


def _dense_proj(x2d, Wmat, bvec):
    M, Cin = x2d.shape
    Cout = Wmat.shape[0]
    TM = 512
    assert M % TM == 0
    return pl.pallas_call(
        _mm_kernel,
        grid=(M // TM,),
        in_specs=[
            pl.BlockSpec((TM, Cin), lambda i: (i, 0)),
            pl.BlockSpec((Cin, Cout), lambda i: (0, 0)),
            pl.BlockSpec((1, Cout), lambda i: (0, 0)),
        ],
        out_specs=pl.BlockSpec((TM, Cout), lambda i: (i, 0)),
        out_shape=jax.ShapeDtypeStruct((M, Cout), jnp.float32),
    )(x2d, Wmat.T, bvec.reshape(1, Cout))


_NPTS = 9
_TB = 64
_CSZ = _TB * _NPTS
_CHUNKS = (128, 128, 128, 128, 64)


def _make_sc_gather(n_rows):
    assert n_rows % (_NW * _TB) == 0
    ch = n_rows // _NW
    nt = ch // _TB

    mesh = plsc.VectorSubcoreMesh(core_axis_name="c", subcore_axis_name="s")

    @functools.partial(
        pl.kernel,
        mesh=mesh,
        compiler_params=pltpu.CompilerParams(use_tc_tiling_on_sc=False),
        out_type=jax.ShapeDtypeStruct((n_rows, 16), jnp.float32),
        scratch_types=[
            pltpu.VMEM((2, 4 * _CSZ), jnp.int32),
            pltpu.VMEM((2, 4, _CSZ + 16), jnp.float32),
            pltpu.VMEM((2, 4 * _CSZ, 16), jnp.float32),
            pltpu.VMEM((2, _TB, 16), jnp.float32),
            pltpu.SemaphoreType.DMA,
            pltpu.SemaphoreType.DMA,
            pltpu.SemaphoreType.DMA,
            pltpu.SemaphoreType.DMA,
        ],
    )
    def sc_gather(tbl, i0, i1, i2, i3, w0, w1, w2, w3, out,
                  idx_v, wgt_v, rows_v, out_v, isem, wsem, gsem, osem):
        wid = lax.axis_index("s") * _NC + lax.axis_index("c")
        base = wid * ch
        idxs = (i0, i1, i2, i3)
        wgts = (w0, w1, w2, w3)

        def issue_idx(t, bf):
            for c in range(4):
                pltpu.async_copy(
                    idxs[c].at[pl.ds((base + t * _TB) * _NPTS, _CSZ)],
                    idx_v.at[bf, pl.ds(c * _CSZ, _CSZ)], isem)

        def wait_idx(bf):
            pltpu.make_async_copy(
                i0.at[pl.ds(0, 4 * _CSZ)], idx_v.at[bf], isem).wait()

        def issue_wgt(t, bf):
            for c in range(4):
                pltpu.async_copy(
                    wgts[c].at[pl.ds((base + t * _TB) * _NPTS, _CSZ)],
                    wgt_v.at[bf, c, pl.ds(0, _CSZ)], wsem)

        def wait_wgt(bf):
            for c in range(4):
                pltpu.make_async_copy(
                    wgts[c].at[pl.ds(0, _CSZ)],
                    wgt_v.at[bf, c, pl.ds(0, _CSZ)], wsem).wait()

        def fire_gathers(bf):
            for c in range(4):
                o = 0
                for sz in _CHUNKS:
                    pltpu.async_copy(
                        tbl.at[idx_v.at[bf, pl.ds(c * _CSZ + o, sz)]],
                        rows_v.at[bf, pl.ds(c * _CSZ + o, sz)], gsem)
                    o += sz

        def wait_gathers(bf):
            pltpu.make_async_copy(
                tbl.at[pl.ds(0, 4 * _CSZ)], rows_v.at[bf], gsem).wait()

        def wait_out(bf):
            pltpu.make_async_copy(
                out_v.at[bf], out.at[pl.ds(0, _TB)], osem).wait()

        issue_idx(0, 0)
        wait_idx(0)
        fire_gathers(0)
        issue_idx(1, 1)
        issue_wgt(0, 0)

        def step(s, carry):
            for b in range(2):
                t = s * 2 + b
                bf = b
                nbf = 1 - b

                @pl.when(t + 1 < nt)
                def _():
                    wait_idx(nbf)
                    fire_gathers(nbf)

                wait_gathers(bf)

                @pl.when(t + 2 < nt)
                def _():
                    issue_idx(t + 2, bf)

                wait_wgt(bf)

                @pl.when(t + 1 < nt)
                def _():
                    issue_wgt(t + 1, nbf)

                @pl.when(t >= 2)
                def _():
                    wait_out(bf)

                def row_body(r, carry2):
                    acc = None
                    for c in range(4):
                        wv = wgt_v[bf, c, pl.ds(r * _NPTS, 16)]
                        b0 = c * _CSZ + r * _NPTS
                        for k in range(_NPTS):
                            term = rows_v[bf, b0 + k, :] * wv[k]
                            acc = term if acc is None else acc + term
                    out_v[bf, r, :] = acc
                    return carry2

                lax.fori_loop(0, _TB, row_body, 0, unroll=2)
                pltpu.async_copy(
                    out_v.at[bf], out.at[pl.ds(base + t * _TB, _TB)], osem)
            return carry

        lax.fori_loop(0, nt // 2, step, 0, unroll=False)
        wait_out(0)
        wait_out(1)

    return sc_gather


_PTM = 512


def _prep_kernel(N, H, W, C, G, Hp, Wp):
    GK = G * _NPTS
    HW = H * W

    def body(x1_ref, lng_ref, lnb_ref, woff_ref, boff_ref, wmsk_ref, bmsk_ref,
             i0, i1, i2, i3, w0, w1, w2, w3):
        x = x1_ref[...]
        mu = jnp.mean(x, axis=1, keepdims=True)
        xc_ = x - mu
        var = jnp.mean(xc_ * xc_, axis=1, keepdims=True)
        xn = xc_ * jax.lax.rsqrt(var + _EPS) * lng_ref[...] + lnb_ref[...]
        xg = 0.5 * xn * (1.0 + jax.lax.erf(xn / np.sqrt(2.0).astype(np.float32)))

        offp = (
            jnp.dot(xg, woff_ref[...], preferred_element_type=jnp.float32)
            + boff_ref[...]
        )
        logits = (
            jnp.dot(xg, wmsk_ref[...], preferred_element_type=jnp.float32)
            + bmsk_ref[...]
        )
        cmax = jnp.max(logits, axis=1, keepdims=True)
        e = jnp.exp(logits - cmax)
        lane = jax.lax.broadcasted_iota(jnp.int32, (GK, G), 0) // _NPTS
        grp = jax.lax.broadcasted_iota(jnp.int32, (GK, G), 1)
        A = (lane == grp).astype(jnp.float32)
        gsum = jnp.dot(e, A, preferred_element_type=jnp.float32)
        denom = jnp.dot(gsum, A.T, preferred_element_type=jnp.float32)
        mask = e / denom

        r_id = pl.program_id(0) * _PTM + jax.lax.broadcasted_iota(
            jnp.int32, (_PTM, 1), 0
        )
        b = r_id // HW
        rem = r_id - b * HW
        h = rem // W
        w_ = rem - h * W
        base_y = (h + _PAD).astype(jnp.float32)
        base_x = (w_ + _PAD).astype(jnp.float32)

        klane = jax.lax.broadcasted_iota(jnp.int32, (1, GK), 1) % _NPTS
        dyv = (klane // _K - (_K - 1) // 2).astype(jnp.float32) * _DIL
        dxv = (klane % _K - (_K - 1) // 2).astype(jnp.float32) * _DIL
        g_lane = jax.lax.broadcasted_iota(jnp.int32, (1, GK), 1) // _NPTS

        py = base_y + dyv + offp[:, :GK]
        px = base_x + dxv + offp[:, GK:]
        y0f = jnp.floor(py)
        x0f = jnp.floor(px)
        wy = py - y0f
        wx = px - x0f
        y0 = y0f.astype(jnp.int32)
        x0 = x0f.astype(jnp.int32)
        tb_base = b * (Hp * Wp * G)

        outs_i = (i0, i1, i2, i3)
        outs_w = (w0, w1, w2, w3)
        corners = (
            (0, 0, (1 - wy) * (1 - wx)),
            (0, 1, (1 - wy) * wx),
            (1, 0, wy * (1 - wx)),
            (1, 1, wy * wx),
        )
        for c, (cy, cx, wgt) in enumerate(corners):
            yi = y0 + cy
            xi = x0 + cx
            valid = (yi >= 0) & (yi < Hp) & (xi >= 0) & (xi < Wp)
            ycl = jnp.clip(yi, 0, Hp - 1)
            xcl = jnp.clip(xi, 0, Wp - 1)
            outs_i[c][...] = tb_base + (ycl * Wp + xcl) * G + g_lane
            outs_w[c][...] = wgt * valid.astype(jnp.float32) * mask

    M = N * H * W
    grid = (M // _PTM,)
    row_spec = pl.BlockSpec((_PTM, C), lambda i: (i, 0))
    fix = lambda shp: pl.BlockSpec(shp, lambda i: tuple(0 for _ in shp))
    out54 = pl.BlockSpec((_PTM, GK), lambda i: (i, 0))

    def run(x1_2d, ln_g, ln_b, W_offp, b_offp, W_maskT, b_mask):
        return pl.pallas_call(
            body,
            grid=grid,
            in_specs=[
                row_spec,
                fix((1, C)),
                fix((1, C)),
                fix((C, 2 * GK)),
                fix((1, 2 * GK)),
                fix((C, GK)),
                fix((1, GK)),
            ],
            out_specs=tuple([out54] * 8),
            out_shape=tuple(
                [jax.ShapeDtypeStruct((M, GK), jnp.int32)] * 4
                + [jax.ShapeDtypeStruct((M, GK), jnp.float32)] * 4
            ),
        )(x1_2d, ln_g.reshape(1, C), ln_b.reshape(1, C), W_offp,
          b_offp.reshape(1, 2 * GK), W_maskT, b_mask.reshape(1, GK))

    return run


def kernel(x, depth, W_in, b_in, W_dw, b_dw, ln_g, ln_b, W_off, b_off, W_mask, b_mask, W_out, b_out):
    N, H, W, C = x.shape
    G, K, PAD = _GROUP, _K, _PAD
    GC = C // G
    Hp, Wp = H + 2 * PAD, W + 2 * PAD
    M = N * H * W

    x_proj = _dense_proj(x.reshape(M, C), W_in, b_in).reshape(N, H, W, C)

    w_hwio = jnp.transpose(W_dw, (2, 3, 1, 0))
    x1 = lax.conv_general_dilated(
        x, w_hwio, window_strides=(1, 1),
        padding=((PAD, PAD), (PAD, PAD)), feature_group_count=C,
        dimension_numbers=("NHWC", "HWIO", "NHWC"))
    x1 = x1 + b_dw

    GK = G * _NPTS
    perm = np.concatenate(
        [np.arange(GK) * 2, np.arange(GK) * 2 + 1]
    )
    W_offp = W_off[perm].T * _OFFSET_SCALE
    b_offp = b_off[perm] * _OFFSET_SCALE

    prep = _prep_kernel(N, H, W, C, G, Hp, Wp)
    i0, i1, i2, i3, w0, w1, w2, w3 = prep(
        x1.reshape(M, C), ln_g, ln_b, W_offp, b_offp, W_mask.T, b_mask
    )
    idxs = [a.reshape(-1) for a in (i0, i1, i2, i3)]
    wgts = [a.reshape(-1) for a in (w0, w1, w2, w3)]

    xp = jnp.pad(x_proj, ((0, 0), (PAD, PAD), (PAD, PAD), (0, 0)))
    tbl = xp.reshape(N * Hp * Wp * G, GC)

    n_rows = M * G
    y2d = _make_sc_gather(n_rows)(tbl, *idxs, *wgts)

    y = y2d.reshape(M, C)
    out = _dense_proj(y, W_out, b_out).reshape(N, H, W, C)
    return (out, depth)
